# baseline (speedup 1.0000x reference)
import functools

import jax
import jax.numpy as jnp
from jax import lax
from jax.experimental import pallas as pl
from jax.experimental.pallas import tpu as pltpu
from jax.experimental.pallas import tpu_sc as plsc

N = 50000
E = 800000
D = 64

NC, NS, L = 2, 16, 16
CB = 64
EPT = 50176
EPAD = NS * EPT
IB = 1024
NPAIR = IB // (2 * CB)
NH = N // NC
ACC_ROWS = 25088
FPT = ACC_ROWS // NS
ZB = 14
ZBD = 112


def _rsqrt16(x):
    xi = plsc.bitcast(x, jnp.int32)
    yi = jnp.full((16,), 0x5F3759DF, jnp.int32) - lax.shift_right_logical(xi, 1)
    y = plsc.bitcast(yi, jnp.float32)
    for _ in range(3):
        y = y * (1.5 - 0.5 * x * y * y)
    return y


def _rsqrt16_fast(x):
    xi = plsc.bitcast(x, jnp.int32)
    yi = jnp.full((16,), 0x5F3759DF, jnp.int32) - lax.shift_right_logical(xi, 1)
    y = plsc.bitcast(yi, jnp.float32)
    for _ in range(2):
        y = y * (1.5 - 0.5 * x * y * y)
    return y


def _sc_edge_body(A, B, iH, jH, lnH, S, DEG,
                  i_v, j_v, a2, b2, m2, di2, ones_v, lnv,
                  zbuf, zdeg, acc, dacc,
                  sa0, sa1, sb0, sb1, sc0, sc1, sd0, sd1):
    c = lax.axis_index("c")
    t = lax.axis_index("s")
    zero16 = jnp.zeros((16,), jnp.float32)

    def zrow(r, _):
        for k in range(4):
            zbuf[r, pl.ds(k * 16, 16)] = zero16
        return 0
    lax.fori_loop(0, ZB, zrow, 0)
    for k in range(ZBD // 16):
        zdeg[pl.ds(k * 16, 16)] = zero16
    for k in range(CB // 16):
        ones_v[pl.ds(k * 16, 16)] = jnp.full((16,), 1.0, jnp.float32)
    pltpu.sync_copy(lnH, lnv)
    lgv = [lnv[0, pl.ds(k * 16, 16)] for k in range(D // 16)]
    lbv = [lnv[1, pl.ds(k * 16, 16)] for k in range(D // 16)]

    def zacc(k, _):
        pltpu.sync_copy(zbuf, acc.at[pl.ds(t * FPT + k * ZB, ZB)])
        return 0
    lax.fori_loop(0, FPT // ZB, zacc, 0)

    def zdacc(k, _):
        pltpu.sync_copy(zdeg, dacc.at[pl.ds(t * FPT + k * ZBD, ZBD)])
        return 0
    lax.fori_loop(0, FPT // ZBD, zdacc, 0)
    plsc.subcore_barrier()

    ebase = t * EPT
    rows_base = lax.iota(jnp.int32, 16)
    sas = (sa0, sa1)
    sbs = (sb0, sb1)
    scs = (sc0, sc1)
    sds = (sd0, sd1)

    def issue(ch, b):
        sl = pl.ds(ch * CB, CB)
        pltpu.async_copy(A.at[i_v.at[sl]], a2.at[pl.ds(b * CB, CB)], sas[b])
        pltpu.async_copy(B.at[j_v.at[sl]], b2.at[pl.ds(b * CB, CB)], sbs[b])

    def gwait(b):
        pltpu.make_async_copy(A.at[pl.ds(0, CB)],
                              a2.at[pl.ds(b * CB, CB)], sas[b]).wait()
        pltpu.make_async_copy(B.at[pl.ds(0, CB)],
                              b2.at[pl.ds(b * CB, CB)], sbs[b]).wait()

    def scat_issue(b):
        sl = pl.ds(b * CB, CB)
        pltpu.async_copy(m2.at[sl], acc.at[di2.at[sl]], scs[b], add=True)
        pltpu.async_copy(ones_v, dacc.at[di2.at[sl]], sds[b], add=True)

    def scat_wait(b):
        sl = pl.ds(b * CB, CB)
        pltpu.make_async_copy(m2.at[sl], acc.at[di2.at[sl]], scs[b]).wait()
        pltpu.make_async_copy(ones_v, dacc.at[di2.at[sl]], sds[b]).wait()

    for k in range(2 * CB // 16):
        di2[pl.ds(k * 16, 16)] = jnp.full((16,), NH, jnp.int32)
    scat_issue(0)
    scat_issue(1)

    def compute(ch, b, gbase0):
        off = ch * CB
        dbase = b * CB
        scat_wait(b)
        for g in range(CB // 16):
            iv16 = i_v[pl.ds(off + g * 16, 16)]
            gid = rows_base + (gbase0 + off + g * 16)
            dii = iv16 - c * NH
            valid = (dii >= 0) & (dii < NH) & (gid < E)
            dii = jnp.where(valid, dii, NH)
            di2[pl.ds(dbase + g * 16, 16)] = dii

        @plsc.parallel_loop(0, CB, 1, unroll=4)
        def _edge(e):
            eb = dbase + e
            h = [a2[eb, pl.ds(k * 16, 16)] + b2[eb, pl.ds(k * 16, 16)]
                 for k in range(D // 16)]
            s = (h[0] + h[1]) + (h[2] + h[3])
            mv = jnp.full((16,), 0.0, jnp.float32) + jnp.sum(s) * (1.0 / D)
            d = [h[k] - mv for k in range(D // 16)]
            q = (d[0] * d[0] + d[1] * d[1]) + (d[2] * d[2] + d[3] * d[3])
            vv = jnp.full((16,), 1e-5, jnp.float32) + jnp.sum(q) * (1.0 / D)
            rv = _rsqrt16(vv)
            for k in range(D // 16):
                y = jnp.maximum(d[k] * rv * lgv[k] + lbv[k], 0.0)
                m2[eb, pl.ds(k * 16, 16)] = y

        scat_issue(b)

    def outer(ob, _):
        pltpu.sync_copy(iH.at[pl.ds(ebase + ob * IB, IB)], i_v)
        pltpu.sync_copy(jH.at[pl.ds(ebase + ob * IB, IB)], j_v)
        gbase0 = ebase + ob * IB
        issue(0, 0)

        def pair(cc, _):
            ch = 2 * cc
            issue(ch + 1, 1)
            gwait(0)
            compute(ch, 0, gbase0)

            @pl.when(cc < NPAIR - 1)
            def _():
                issue(ch + 2, 0)
            gwait(1)
            compute(ch + 1, 1, gbase0)
            return 0
        lax.fori_loop(0, NPAIR, pair, 0)
        return 0
    lax.fori_loop(0, EPT // IB, outer, 0)
    scat_wait(0)
    scat_wait(1)
    plsc.subcore_barrier()

    last = NH - (NS - 1) * FPT

    @pl.when(t < NS - 1)
    def _():
        pltpu.sync_copy(acc.at[pl.ds(t * FPT, FPT)],
                        S.at[pl.ds(c * NH + t * FPT, FPT)])
        pltpu.sync_copy(dacc.at[pl.ds(t * FPT, FPT)],
                        DEG.at[pl.ds(c * NH + t * FPT, FPT)])

    @pl.when(t == NS - 1)
    def _():
        pltpu.sync_copy(acc.at[pl.ds((NS - 1) * FPT, last)],
                        S.at[pl.ds(c * NH + (NS - 1) * FPT, last)])
        pltpu.sync_copy(dacc.at[pl.ds((NS - 1) * FPT, last)],
                        DEG.at[pl.ds(c * NH + (NS - 1) * FPT, last)])


_sc_edge = functools.partial(
    pl.kernel,
    out_type=[jax.ShapeDtypeStruct((N, D), jnp.float32),
              jax.ShapeDtypeStruct((N,), jnp.float32)],
    mesh=plsc.VectorSubcoreMesh(core_axis_name="c", subcore_axis_name="s",
                                num_cores=NC, num_subcores=NS),
    compiler_params=pltpu.CompilerParams(needs_layout_passes=False,
                                         use_tc_tiling_on_sc=False),
    scratch_types=[
        pltpu.VMEM((IB,), jnp.int32),
        pltpu.VMEM((IB,), jnp.int32),
        pltpu.VMEM((2 * CB, D), jnp.float32),
        pltpu.VMEM((2 * CB, D), jnp.float32),
        pltpu.VMEM((2 * CB, D), jnp.float32),
        pltpu.VMEM((2 * CB,), jnp.int32),
        pltpu.VMEM((CB,), jnp.float32),
        pltpu.VMEM((2, D), jnp.float32),
        pltpu.VMEM((ZB, D), jnp.float32),
        pltpu.VMEM((ZBD,), jnp.float32),
        pltpu.VMEM_SHARED((ACC_ROWS, D), jnp.float32),
        pltpu.VMEM_SHARED((ACC_ROWS,), jnp.float32),
        pltpu.SemaphoreType.DMA,
        pltpu.SemaphoreType.DMA,
        pltpu.SemaphoreType.DMA,
        pltpu.SemaphoreType.DMA,
        pltpu.SemaphoreType.DMA,
        pltpu.SemaphoreType.DMA,
        pltpu.SemaphoreType.DMA,
        pltpu.SemaphoreType.DMA,
    ],
)(_sc_edge_body)


BLK = 1000
GRID = N // BLK


def _mm(x, w):
    return lax.dot_general(x, w, (((1,), (1,)), ((), ())),
                           precision=lax.Precision.HIGHEST,
                           preferred_element_type=jnp.float32)


def _lnb(x, g, b):
    m = jnp.mean(x, axis=-1, keepdims=True)
    v = jnp.mean((x - m) ** 2, axis=-1, keepdims=True)
    return (x - m) / jnp.sqrt(v + 1e-5) * g + b


def _row_spec(w):
    return pl.BlockSpec((BLK, w), lambda i: (i, 0))


def _full_spec(shape):
    nd = len(shape)
    return pl.BlockSpec(shape, lambda i: (0,) * nd)


def _tc_call(body, row_ins, weight_ins, n_out, out_w=D):
    in_specs = ([_row_spec(x.shape[1]) for x in row_ins]
                + [_full_spec(w.shape) for w in weight_ins])
    out_shape = [jax.ShapeDtypeStruct((N, w), jnp.float32)
                 for w in ([out_w] * n_out if isinstance(out_w, int) else out_w)]
    out_specs = [pl.BlockSpec((BLK, s.shape[1]), lambda i: (i, 0))
                 for s in out_shape]
    if n_out == 1:
        out_shape, out_specs = out_shape[0], out_specs[0]
    return pl.pallas_call(
        body, grid=(GRID,), in_specs=in_specs,
        out_specs=out_specs, out_shape=out_shape,
    )(*row_ins, *weight_ins)


def _embed_body(grp, cons, var,
                glg, glb, gW, gb, clg, clb, cW1, cb1, cW2, cb2,
                vlg, vlb, vW1, vb1, vW2, vb2, c_out, v_out):
    g = jax.nn.relu(_mm(_lnb(grp[...], glg[...], glb[...]), gW[...]) + gb[...])
    c = _lnb(cons[...], clg[...], clb[...])
    c = jax.nn.relu(_mm(c, cW1[...]) + cb1[...])
    c_out[...] = jax.nn.relu(_mm(c, cW2[...]) + cb2[...])
    v = _lnb(var[...], vlg[...], vlb[...])
    v = jax.nn.relu(_mm(v, vW1[...]) + vb1[...])
    v_out[...] = jax.nn.relu(_mm(v, vW2[...]) + vb2[...]) + g


def _pre_body(right, left, Wl, bl, Wr, a_out, b_out):
    a_out[...] = _mm(right[...], Wl[...]) + bl[...]
    b_out[...] = _mm(left[...], Wr[...])


def _post_body(S, deg, right, Wf, bf, pg, pb, Wo1a, Wo1b, bo1, Wo2, bo2, out):
    agg = _mm(S[...], Wf[...]) + deg[...] * bf[...]
    tn = _lnb(agg, pg[...], pb[...])
    u = jax.nn.relu(_mm(tn, Wo1a[...]) + _mm(right[...], Wo1b[...]) + bo1[...])
    out[...] = _mm(u, Wo2[...]) + bo2[...]


def _head_body(v, oW1, ob1, oW2, out):
    u = jax.nn.relu(_mm(v[...], oW1[...]) + ob1[...])
    out[...] = _mm(u, oW2[...])


def _r2(x):
    return x.reshape(1, -1)


def kernel(constraint_features, edge_indices, edge_features,
           variable_features, group_features, params):
    del edge_features
    p = params
    pad = jnp.zeros((EPAD - E,), jnp.int32)
    i_c = jnp.concatenate([edge_indices[0], pad])
    i_v = jnp.concatenate([edge_indices[1], pad])

    c0, v0 = _tc_call(
        _embed_body,
        [group_features, constraint_features, variable_features],
        [_r2(p['g_lg']), _r2(p['g_lb']), p['g_W'], _r2(p['g_b']),
         _r2(p['c_lg']), _r2(p['c_lb']), p['c_W1'], _r2(p['c_b1']),
         p['c_W2'], _r2(p['c_b2']),
         _r2(p['v_lg']), _r2(p['v_lb']), p['v_W1'], _r2(p['v_b1']),
         p['v_W2'], _r2(p['v_b2'])],
        n_out=2)

    e_const = p['e_lb'][0]

    def conv_round(q, left, right, idx_dst, idx_src):
        bl = _r2(q['bl'] + e_const * q['We'][:, 0])
        A, B = _tc_call(_pre_body, [right, left],
                        [q['Wl'], bl, q['Wr']], n_out=2)
        ln2 = jnp.stack([q['lg'], q['lb']])
        S, deg = _sc_edge(A, B, idx_dst, idx_src, ln2)
        return _tc_call(
            _post_body, [S, deg.reshape(-1, 1), right],
            [q['Wf'], _r2(q['bf']), _r2(q['pg']), _r2(q['pb']),
             q['Wo1'][:, :D], q['Wo1'][:, D:], _r2(q['bo1']),
             q['Wo2'], _r2(q['bo2'])],
            n_out=1)

    c1 = conv_round(p['conv1'], v0, c0, i_c, i_v)
    v1 = conv_round(p['conv2'], c1, v0, i_v, i_c)
    c2 = conv_round(p['conv3'], v1, c1, i_c, i_v)
    v2 = conv_round(p['conv4'], c2, v1, i_v, i_c)

    out = _tc_call(_head_body, [v2],
                   [p['o_W1'], _r2(p['o_b1']), p['o_W2']],
                   n_out=1, out_w=[1])
    return out.reshape(-1)


# speedup vs baseline: 1.4073x; 1.4073x over previous
import functools

import jax
import jax.numpy as jnp
from jax import lax
from jax.experimental import pallas as pl
from jax.experimental.pallas import tpu as pltpu
from jax.experimental.pallas import tpu_sc as plsc

N = 50000
E = 800000
D = 64

NC, NS, L = 2, 16, 16
CB = 64
EPT = 50176
EPAD = NS * EPT
IB = 1024
NPAIR = IB // (2 * CB)
NH = N // NC
ACC_ROWS = 25088
FPT = ACC_ROWS // NS
ZB = 14
ZBD = 112
FB = 1024
REG = EPT + 2176
EPP = NS * REG
SENT_LO = NH
SENT_HI = 0


def _rsqrt16(x):
    xi = plsc.bitcast(x, jnp.int32)
    yi = jnp.full((16,), 0x5F3759DF, jnp.int32) - lax.shift_right_logical(xi, 1)
    y = plsc.bitcast(yi, jnp.float32)
    for _ in range(3):
        y = y * (1.5 - 0.5 * x * y * y)
    return y


def _rsqrt16_fast(x):
    xi = plsc.bitcast(x, jnp.int32)
    yi = jnp.full((16,), 0x5F3759DF, jnp.int32) - lax.shift_right_logical(xi, 1)
    y = plsc.bitcast(yi, jnp.float32)
    for _ in range(2):
        y = y * (1.5 - 0.5 * x * y * y)
    return y


def _al8(x):
    return pl.multiple_of(x, 8)


def _sc_part_body(DC, DV, PD0, PS0, PD1, PS1, CNT,
                  i_v, j_v, ld, ls, hd, hs, cnt_buf):
    c = lax.axis_index("c")
    t = lax.axis_index("s")
    rows_base = lax.iota(jnp.int32, 16)
    zero = jnp.int32(0)

    def run(DH, SH, PD, PS):
        ebase = t * EPT
        rbase = t * REG
        rend = (t + 1) * REG

        def blk(ob, carry):
            pltpu.sync_copy(DH.at[pl.ds(ebase + ob * IB, IB)], i_v)
            pltpu.sync_copy(SH.at[pl.ds(ebase + ob * IB, IB)], j_v)
            gb = ebase + ob * IB

            def grp(g, cy):
                lom, lof, him, hif = cy
                dv = i_v[pl.ds(g * 16, 16)]
                sv = j_v[pl.ds(g * 16, 16)]
                inb = (rows_base + (gb + g * 16)) < E
                islo = dv < NH
                mlo = islo & inb
                mhi = (~islo) & inb
                plsc.store_compressed(ld.at[pl.ds(lom, 16)], dv, mask=mlo)
                plsc.store_compressed(ls.at[pl.ds(lom, 16)], sv, mask=mlo)
                plsc.store_compressed(hd.at[pl.ds(him, 16)], dv, mask=mhi)
                plsc.store_compressed(hs.at[pl.ds(him, 16)], sv, mask=mhi)
                cl = plsc.all_reduce_population_count(mlo)[0]
                chi = plsc.all_reduce_population_count(mhi)[0]
                lom = lom + cl
                him = him + chi

                @pl.when(lom >= FB)
                def _():
                    pltpu.sync_copy(ld.at[pl.ds(0, FB)],
                                    PD.at[pl.ds(_al8(rbase + lof), FB)])
                    pltpu.sync_copy(ls.at[pl.ds(0, FB)],
                                    PS.at[pl.ds(_al8(rbase + lof), FB)])
                    ld[pl.ds(0, 16)] = ld[pl.ds(FB, 16)]
                    ls[pl.ds(0, 16)] = ls[pl.ds(FB, 16)]
                lof = jnp.where(lom >= FB, lof + FB, lof)
                lom = jnp.where(lom >= FB, lom - FB, lom)

                @pl.when(him >= FB)
                def _():
                    pltpu.sync_copy(hd.at[pl.ds(0, FB)],
                                    PD.at[pl.ds(_al8(rend - hif - FB), FB)])
                    pltpu.sync_copy(hs.at[pl.ds(0, FB)],
                                    PS.at[pl.ds(_al8(rend - hif - FB), FB)])
                    hd[pl.ds(0, 16)] = hd[pl.ds(FB, 16)]
                    hs[pl.ds(0, 16)] = hs[pl.ds(FB, 16)]
                hif = jnp.where(him >= FB, hif + FB, hif)
                him = jnp.where(him >= FB, him - FB, him)
                return (lom, lof, him, hif)
            return lax.fori_loop(0, IB // 16, grp, carry)
        lom, lof, him, hif = lax.fori_loop(0, EPT // IB, blk,
                                           (zero, zero, zero, zero))

        slo = jnp.full((16,), SENT_LO, jnp.int32)
        shi = jnp.full((16,), SENT_HI, jnp.int32)
        szr = jnp.full((16,), 0, jnp.int32)

        ld[pl.ds(lom, 16)] = slo
        ls[pl.ds(lom, 16)] = szr
        n16 = lax.shift_right_logical(lom + 15, 4)

        def ltail(k, _):
            pltpu.sync_copy(ld.at[pl.ds(k * 16, 16)],
                            PD.at[pl.ds(_al8(rbase + lof + k * 16), 16)])
            pltpu.sync_copy(ls.at[pl.ds(k * 16, 16)],
                            PS.at[pl.ds(_al8(rbase + lof + k * 16), 16)])
            return 0
        lax.fori_loop(0, n16, ltail, 0)

        def lsfill(k, _):
            ld[pl.ds(k * 16, 16)] = slo
            ls[pl.ds(k * 16, 16)] = szr
            return 0
        lax.fori_loop(0, FB // 16, lsfill, 0)
        lo16 = lof + n16 * 16
        pltpu.sync_copy(ld.at[pl.ds(0, FB)], PD.at[pl.ds(_al8(rbase + lo16), FB)])
        pltpu.sync_copy(ls.at[pl.ds(0, FB)], PS.at[pl.ds(_al8(rbase + lo16), FB)])

        hd[pl.ds(him, 16)] = shi
        hs[pl.ds(him, 16)] = szr
        n16h = lax.shift_right_logical(him + 15, 4)

        def htail(k, _):
            pltpu.sync_copy(hd.at[pl.ds(k * 16, 16)],
                            PD.at[pl.ds(_al8(rend - hif - (k + 1) * 16), 16)])
            pltpu.sync_copy(hs.at[pl.ds(k * 16, 16)],
                            PS.at[pl.ds(_al8(rend - hif - (k + 1) * 16), 16)])
            return 0
        lax.fori_loop(0, n16h, htail, 0)

        def hsfill(k, _):
            hd[pl.ds(k * 16, 16)] = shi
            hs[pl.ds(k * 16, 16)] = szr
            return 0
        lax.fori_loop(0, FB // 16, hsfill, 0)
        hi16 = hif + n16h * 16
        pltpu.sync_copy(hd.at[pl.ds(0, FB)],
                        PD.at[pl.ds(_al8(rend - hi16 - FB), FB)])
        pltpu.sync_copy(hs.at[pl.ds(0, FB)],
                        PS.at[pl.ds(_al8(rend - hi16 - FB), FB)])

        cv = jnp.where(rows_base == 0, jnp.full((16,), 0, jnp.int32) + lo16,
                       jnp.where(rows_base == 1,
                                 jnp.full((16,), 0, jnp.int32) + hi16, 0))
        cnt_buf[pl.ds(0, 16)] = cv
        pltpu.sync_copy(cnt_buf.at[pl.ds(0, 8)], CNT.at[c, t])

    @pl.when(c == 0)
    def _():
        run(DC, DV, PD0, PS0)

    @pl.when(c == 1)
    def _():
        run(DV, DC, PD1, PS1)


_sc_part = functools.partial(
    pl.kernel,
    out_type=[jax.ShapeDtypeStruct((EPP,), jnp.int32),
              jax.ShapeDtypeStruct((EPP,), jnp.int32),
              jax.ShapeDtypeStruct((EPP,), jnp.int32),
              jax.ShapeDtypeStruct((EPP,), jnp.int32),
              jax.ShapeDtypeStruct((NC, NS, 8), jnp.int32)],
    mesh=plsc.VectorSubcoreMesh(core_axis_name="c", subcore_axis_name="s",
                                num_cores=NC, num_subcores=NS),
    compiler_params=pltpu.CompilerParams(needs_layout_passes=False,
                                         use_tc_tiling_on_sc=False),
    scratch_types=[
        pltpu.VMEM((IB,), jnp.int32),
        pltpu.VMEM((IB,), jnp.int32),
        pltpu.VMEM((FB + 80,), jnp.int32),
        pltpu.VMEM((FB + 80,), jnp.int32),
        pltpu.VMEM((FB + 80,), jnp.int32),
        pltpu.VMEM((FB + 80,), jnp.int32),
        pltpu.VMEM((16,), jnp.int32),
    ],
)(_sc_part_body)


def _sc_edge_body(A, B, PD, PS, CNTd, lnH, S, DEG,
                  i_v, j_v, a2, b2, m2, di2, ones_v, lnv, cnt_v,
                  zbuf, zdeg, acc, dacc,
                  sa0, sa1, sb0, sb1, sc0, sc1, sd0, sd1):
    c = lax.axis_index("c")
    t = lax.axis_index("s")
    zero16 = jnp.zeros((16,), jnp.float32)

    def zrow(r, _):
        for k in range(4):
            zbuf[r, pl.ds(k * 16, 16)] = zero16
        return 0
    lax.fori_loop(0, ZB, zrow, 0)
    for k in range(ZBD // 16):
        zdeg[pl.ds(k * 16, 16)] = zero16
    for k in range(CB // 16):
        ones_v[pl.ds(k * 16, 16)] = jnp.full((16,), 1.0, jnp.float32)
    pltpu.sync_copy(lnH, lnv)
    lgv = [lnv[0, pl.ds(k * 16, 16)] for k in range(D // 16)]
    lbv = [lnv[1, pl.ds(k * 16, 16)] for k in range(D // 16)]

    def zacc(k, _):
        pltpu.sync_copy(zbuf, acc.at[pl.ds(t * FPT + k * ZB, ZB)])
        return 0
    lax.fori_loop(0, FPT // ZB, zacc, 0)

    def zdacc(k, _):
        pltpu.sync_copy(zdeg, dacc.at[pl.ds(t * FPT + k * ZBD, ZBD)])
        return 0
    lax.fori_loop(0, FPT // ZBD, zdacc, 0)
    plsc.subcore_barrier()

    rows_base = lax.iota(jnp.int32, 16)
    sas = (sa0, sa1)
    sbs = (sb0, sb1)
    scs = (sc0, sc1)
    sds = (sd0, sd1)

    def issue(ch, b):
        sl = pl.ds(ch * CB, CB)
        pltpu.async_copy(A.at[i_v.at[sl]], a2.at[pl.ds(b * CB, CB)], sas[b])
        pltpu.async_copy(B.at[j_v.at[sl]], b2.at[pl.ds(b * CB, CB)], sbs[b])

    def gwait(b):
        pltpu.make_async_copy(A.at[pl.ds(0, CB)],
                              a2.at[pl.ds(b * CB, CB)], sas[b]).wait()
        pltpu.make_async_copy(B.at[pl.ds(0, CB)],
                              b2.at[pl.ds(b * CB, CB)], sbs[b]).wait()

    def scat_issue(b):
        sl = pl.ds(b * CB, CB)
        pltpu.async_copy(m2.at[sl], acc.at[di2.at[sl]], scs[b], add=True)
        pltpu.async_copy(ones_v, dacc.at[di2.at[sl]], sds[b], add=True)

    def scat_wait(b):
        sl = pl.ds(b * CB, CB)
        pltpu.make_async_copy(m2.at[sl], acc.at[di2.at[sl]], scs[b]).wait()
        pltpu.make_async_copy(ones_v, dacc.at[di2.at[sl]], sds[b]).wait()

    for k in range(2 * CB // 16):
        di2[pl.ds(k * 16, 16)] = jnp.full((16,), NH, jnp.int32)
    scat_issue(0)
    scat_issue(1)

    def compute(ch, b):
        off = ch * CB
        dbase = b * CB
        scat_wait(b)
        for g in range(CB // 16):
            iv16 = i_v[pl.ds(off + g * 16, 16)]
            dii = iv16 - c * NH
            valid = (dii >= 0) & (dii < NH)
            dii = jnp.where(valid, dii, NH)
            di2[pl.ds(dbase + g * 16, 16)] = dii

        @plsc.parallel_loop(0, CB, 1, unroll=4)
        def _edge(e):
            eb = dbase + e
            h = [a2[eb, pl.ds(k * 16, 16)] + b2[eb, pl.ds(k * 16, 16)]
                 for k in range(D // 16)]
            s = (h[0] + h[1]) + (h[2] + h[3])
            mv = jnp.full((16,), 0.0, jnp.float32) + jnp.sum(s) * (1.0 / D)
            d = [h[k] - mv for k in range(D // 16)]
            q = (d[0] * d[0] + d[1] * d[1]) + (d[2] * d[2] + d[3] * d[3])
            vv = jnp.full((16,), 1e-5, jnp.float32) + jnp.sum(q) * (1.0 / D)
            rv = _rsqrt16(vv)
            for k in range(D // 16):
                y = jnp.maximum(d[k] * rv * lgv[k] + lbv[k], 0.0)
                m2[eb, pl.ds(k * 16, 16)] = y

        scat_issue(b)

    pltpu.sync_copy(CNTd.at[t], cnt_v.at[pl.ds(0, 8)])
    cw = cnt_v[pl.ds(0, 16)]
    cnt16 = jnp.where(c == 0, cw[0], cw[1])
    nblk = lax.shift_right_logical(cnt16 + (IB - 1), 10)
    base0 = jnp.where(c == 0, t * REG, (t + 1) * REG - nblk * IB)

    def outer(ob, _):
        pltpu.sync_copy(PD.at[pl.ds(_al8(base0 + ob * IB), IB)], i_v)
        pltpu.sync_copy(PS.at[pl.ds(_al8(base0 + ob * IB), IB)], j_v)
        issue(0, 0)

        def pair(cc, _):
            ch = 2 * cc
            issue(ch + 1, 1)
            gwait(0)
            compute(ch, 0)

            @pl.when(cc < NPAIR - 1)
            def _():
                issue(ch + 2, 0)
            gwait(1)
            compute(ch + 1, 1)
            return 0
        lax.fori_loop(0, NPAIR, pair, 0)
        return 0
    lax.fori_loop(0, nblk, outer, 0)
    scat_wait(0)
    scat_wait(1)
    plsc.subcore_barrier()

    last = NH - (NS - 1) * FPT

    @pl.when(t < NS - 1)
    def _():
        pltpu.sync_copy(acc.at[pl.ds(t * FPT, FPT)],
                        S.at[pl.ds(c * NH + t * FPT, FPT)])
        pltpu.sync_copy(dacc.at[pl.ds(t * FPT, FPT)],
                        DEG.at[pl.ds(c * NH + t * FPT, FPT)])

    @pl.when(t == NS - 1)
    def _():
        pltpu.sync_copy(acc.at[pl.ds((NS - 1) * FPT, last)],
                        S.at[pl.ds(c * NH + (NS - 1) * FPT, last)])
        pltpu.sync_copy(dacc.at[pl.ds((NS - 1) * FPT, last)],
                        DEG.at[pl.ds(c * NH + (NS - 1) * FPT, last)])


_sc_edge = functools.partial(
    pl.kernel,
    out_type=[jax.ShapeDtypeStruct((N, D), jnp.float32),
              jax.ShapeDtypeStruct((N,), jnp.float32)],
    mesh=plsc.VectorSubcoreMesh(core_axis_name="c", subcore_axis_name="s",
                                num_cores=NC, num_subcores=NS),
    compiler_params=pltpu.CompilerParams(needs_layout_passes=False,
                                         use_tc_tiling_on_sc=False),
    scratch_types=[
        pltpu.VMEM((IB,), jnp.int32),
        pltpu.VMEM((IB,), jnp.int32),
        pltpu.VMEM((2 * CB, D), jnp.float32),
        pltpu.VMEM((2 * CB, D), jnp.float32),
        pltpu.VMEM((2 * CB, D), jnp.float32),
        pltpu.VMEM((2 * CB,), jnp.int32),
        pltpu.VMEM((CB,), jnp.float32),
        pltpu.VMEM((2, D), jnp.float32),
        pltpu.VMEM((16,), jnp.int32),
        pltpu.VMEM((ZB, D), jnp.float32),
        pltpu.VMEM((ZBD,), jnp.float32),
        pltpu.VMEM_SHARED((ACC_ROWS, D), jnp.float32),
        pltpu.VMEM_SHARED((ACC_ROWS,), jnp.float32),
        pltpu.SemaphoreType.DMA,
        pltpu.SemaphoreType.DMA,
        pltpu.SemaphoreType.DMA,
        pltpu.SemaphoreType.DMA,
        pltpu.SemaphoreType.DMA,
        pltpu.SemaphoreType.DMA,
        pltpu.SemaphoreType.DMA,
        pltpu.SemaphoreType.DMA,
    ],
)(_sc_edge_body)


BLK = 1000
GRID = N // BLK


def _mm(x, w):
    return lax.dot_general(x, w, (((1,), (1,)), ((), ())),
                           preferred_element_type=jnp.float32)


def _lnb(x, g, b):
    m = jnp.mean(x, axis=-1, keepdims=True)
    v = jnp.mean((x - m) ** 2, axis=-1, keepdims=True)
    return (x - m) / jnp.sqrt(v + 1e-5) * g + b


def _row_spec(w):
    return pl.BlockSpec((BLK, w), lambda i: (i, 0))


def _full_spec(shape):
    nd = len(shape)
    return pl.BlockSpec(shape, lambda i: (0,) * nd)


def _tc_call(body, row_ins, weight_ins, n_out, out_w=D):
    in_specs = ([_row_spec(x.shape[1]) for x in row_ins]
                + [_full_spec(w.shape) for w in weight_ins])
    out_shape = [jax.ShapeDtypeStruct((N, w), jnp.float32)
                 for w in ([out_w] * n_out if isinstance(out_w, int) else out_w)]
    out_specs = [pl.BlockSpec((BLK, s.shape[1]), lambda i: (i, 0))
                 for s in out_shape]
    if n_out == 1:
        out_shape, out_specs = out_shape[0], out_specs[0]
    return pl.pallas_call(
        body, grid=(GRID,), in_specs=in_specs,
        out_specs=out_specs, out_shape=out_shape,
    )(*row_ins, *weight_ins)


def _embed_body(grp, cons, var,
                glg, glb, gW, gb, clg, clb, cW1, cb1, cW2, cb2,
                vlg, vlb, vW1, vb1, vW2, vb2, c_out, v_out):
    g = jax.nn.relu(_mm(_lnb(grp[...], glg[...], glb[...]), gW[...]) + gb[...])
    c = _lnb(cons[...], clg[...], clb[...])
    c = jax.nn.relu(_mm(c, cW1[...]) + cb1[...])
    c_out[...] = jax.nn.relu(_mm(c, cW2[...]) + cb2[...])
    v = _lnb(var[...], vlg[...], vlb[...])
    v = jax.nn.relu(_mm(v, vW1[...]) + vb1[...])
    v_out[...] = jax.nn.relu(_mm(v, vW2[...]) + vb2[...]) + g


def _pre_body(right, left, Wl, bl, Wr, a_out, b_out):
    a_out[...] = _mm(right[...], Wl[...]) + bl[...]
    b_out[...] = _mm(left[...], Wr[...])


def _post_body(S, deg, right, Wf, bf, pg, pb, Wo1a, Wo1b, bo1, Wo2, bo2, out):
    agg = _mm(S[...], Wf[...]) + deg[...] * bf[...]
    tn = _lnb(agg, pg[...], pb[...])
    u = jax.nn.relu(_mm(tn, Wo1a[...]) + _mm(right[...], Wo1b[...]) + bo1[...])
    out[...] = _mm(u, Wo2[...]) + bo2[...]


def _head_body(v, oW1, ob1, oW2, out):
    u = jax.nn.relu(_mm(v[...], oW1[...]) + ob1[...])
    out[...] = _mm(u, oW2[...])


def _r2(x):
    return x.reshape(1, -1)


def kernel(constraint_features, edge_indices, edge_features,
           variable_features, group_features, params):
    del edge_features
    p = params
    pad = jnp.zeros((EPAD - E,), jnp.int32)
    i_c = jnp.concatenate([edge_indices[0], pad])
    i_v = jnp.concatenate([edge_indices[1], pad])
    pd_c, ps_c, pd_v, ps_v, cnt = _sc_part(i_c, i_v)
    cnt_c, cnt_v_ = cnt[0], cnt[1]

    c0, v0 = _tc_call(
        _embed_body,
        [group_features, constraint_features, variable_features],
        [_r2(p['g_lg']), _r2(p['g_lb']), p['g_W'], _r2(p['g_b']),
         _r2(p['c_lg']), _r2(p['c_lb']), p['c_W1'], _r2(p['c_b1']),
         p['c_W2'], _r2(p['c_b2']),
         _r2(p['v_lg']), _r2(p['v_lb']), p['v_W1'], _r2(p['v_b1']),
         p['v_W2'], _r2(p['v_b2'])],
        n_out=2)

    e_const = p['e_lb'][0]

    def conv_round(q, left, right, pd, ps, cn):
        bl = _r2(q['bl'] + e_const * q['We'][:, 0])
        A, B = _tc_call(_pre_body, [right, left],
                        [q['Wl'], bl, q['Wr']], n_out=2)
        ln2 = jnp.stack([q['lg'], q['lb']])
        S, deg = _sc_edge(A, B, pd, ps, cn, ln2)
        return _tc_call(
            _post_body, [S, deg.reshape(-1, 1), right],
            [q['Wf'], _r2(q['bf']), _r2(q['pg']), _r2(q['pb']),
             q['Wo1'][:, :D], q['Wo1'][:, D:], _r2(q['bo1']),
             q['Wo2'], _r2(q['bo2'])],
            n_out=1)

    c1 = conv_round(p['conv1'], v0, c0, pd_c, ps_c, cnt_c)
    v1 = conv_round(p['conv2'], c1, v0, pd_v, ps_v, cnt_v_)
    c2 = conv_round(p['conv3'], v1, c1, pd_c, ps_c, cnt_c)
    v2 = conv_round(p['conv4'], c2, v1, pd_v, ps_v, cnt_v_)

    out = _tc_call(_head_body, [v2],
                   [p['o_W1'], _r2(p['o_b1']), p['o_W2']],
                   n_out=1, out_w=[1])
    return out.reshape(-1)
